# baseline (speedup 1.0000x reference)
import jax
import jax.numpy as jnp
from jax.experimental import pallas as pl
from jax.experimental.pallas import tpu as pltpu

_R, _C = 160, 128
_P = _R * _C
_G = _R // 8
_MOUT = 200
_SELR = 2
_BIG = 2**30


def _tile_iota():
    return (jax.lax.broadcasted_iota(jnp.int32, (8, _C), 0) * _C
            + jax.lax.broadcasted_iota(jnp.int32, (8, _C), 1))


def _argmax_tree(ws):
    ti = _tile_iota()
    pairs = [(ws[pl.ds(g * 8, 8), :], ti + g * 1024) for g in range(_G)]
    while len(pairs) > 1:
        nxt = []
        for j in range(0, len(pairs) - 1, 2):
            (am, ai), (bm, bi) = pairs[j], pairs[j + 1]
            take_b = bm > am
            nxt.append((jnp.where(take_b, bm, am), jnp.where(take_b, bi, ai)))
        if len(pairs) % 2:
            nxt.append(pairs[-1])
        pairs = nxt
    return pairs[0]


def _nms_kernel(thr_ref, x1, y1, x2, y2, s, sel_ref, num_ref, ws, ar):
    iou_thr = thr_ref[0, 0]
    score_thr = thr_ref[1, 0]
    ws[...] = jnp.where(s[...] > score_thr, s[...], -jnp.inf)
    ar[...] = (x2[...] - x1[...]) * (y2[...] - y1[...])

    ti = _tile_iota()
    seli = (jax.lax.broadcasted_iota(jnp.int32, (_SELR, _C), 0) * _C
            + jax.lax.broadcasted_iota(jnp.int32, (_SELR, _C), 1))
    m_v0, i_v0 = _argmax_tree(ws)

    def cond(carry):
        num, stop = carry[0], carry[1]
        return jnp.logical_and(num < _MOUT, jnp.logical_not(stop))

    def body(carry):
        num, stop, sel, sx1, sy1, sx2, sy2, sa, m_v, i_v = carry
        m = jnp.max(m_v)
        valid = m > -jnp.inf
        idx = jnp.min(jnp.where(m_v == m, i_v, _BIG))
        gi = idx // 1024
        base = pl.multiple_of(gi * 8, 8)
        pick = ti == (idx - gi * 1024)
        zero = jnp.float32(0.0)
        tx1 = x1[pl.ds(base, 8), :]
        ty1 = y1[pl.ds(base, 8), :]
        tx2 = x2[pl.ds(base, 8), :]
        ty2 = y2[pl.ds(base, 8), :]
        tar = ar[pl.ds(base, 8), :]
        b0 = jnp.sum(jnp.where(pick, tx1, zero))
        b1 = jnp.sum(jnp.where(pick, ty1, zero))
        b2 = jnp.sum(jnp.where(pick, tx2, zero))
        b3 = jnp.sum(jnp.where(pick, ty2, zero))
        a = jnp.sum(jnp.where(pick, tar, zero))
        xx1 = jnp.maximum(b0, sx1)
        yy1 = jnp.maximum(b1, sy1)
        xx2 = jnp.minimum(b2, sx2)
        yy2 = jnp.minimum(b3, sy2)
        inter = (jnp.clip(xx2 - xx1, 0.0, None)
                 * jnp.clip(yy2 - yy1, 0.0, None))
        union = jnp.maximum(a + sa - inter, 1e-6)
        iou = inter / union
        hit = (iou >= iou_thr) & (ti < num)
        accepted = valid & jnp.logical_not(jnp.any(hit))
        slot = accepted & (ti == num)
        sel = jnp.where(accepted & (seli == num), idx, sel)
        sx1 = jnp.where(slot, b0, sx1)
        sy1 = jnp.where(slot, b1, sy1)
        sx2 = jnp.where(slot, b2, sx2)
        sy2 = jnp.where(slot, b3, sy2)
        sa = jnp.where(slot, a, sa)
        num = num + accepted.astype(jnp.int32)
        wtile = ws[pl.ds(base, 8), :]
        ws[pl.ds(base, 8), :] = jnp.where(valid & pick, -jnp.inf, wtile)
        m_v, i_v = _argmax_tree(ws)
        stop = jnp.logical_not(valid)
        return (num, stop, sel, sx1, sy1, sx2, sy2, sa, m_v, i_v)

    zf = jnp.zeros((8, _C), jnp.float32)
    carry = (jnp.int32(0), jnp.bool_(False),
             jnp.zeros((_SELR, _C), jnp.int32), zf, zf, zf, zf, zf,
             m_v0, i_v0)
    carry = jax.lax.while_loop(cond, body, carry)
    sel_ref[...] = carry[2]
    num_ref[0, 0] = carry[0]


def kernel(boxes, scores, max_output_size, iou_threshold, scores_threshold):
    boxes = boxes.astype(jnp.float32)
    scores = scores.astype(jnp.float32)
    n = boxes.shape[0]
    pad = _P - n
    bx = jnp.pad(boxes, ((0, pad), (0, 0)))
    planes = bx.T.reshape(4, _R, _C)
    s = jnp.pad(scores, (0, pad), constant_values=-jnp.inf).reshape(_R, _C)
    thr = jnp.stack([jnp.asarray(iou_threshold, jnp.float32),
                     jnp.asarray(scores_threshold, jnp.float32)]).reshape(2, 1)

    sel_m, num_m = pl.pallas_call(
        _nms_kernel,
        in_specs=[
            pl.BlockSpec(memory_space=pltpu.SMEM),
            pl.BlockSpec(memory_space=pltpu.VMEM),
            pl.BlockSpec(memory_space=pltpu.VMEM),
            pl.BlockSpec(memory_space=pltpu.VMEM),
            pl.BlockSpec(memory_space=pltpu.VMEM),
            pl.BlockSpec(memory_space=pltpu.VMEM),
        ],
        out_specs=[
            pl.BlockSpec(memory_space=pltpu.VMEM),
            pl.BlockSpec(memory_space=pltpu.SMEM),
        ],
        out_shape=[
            jax.ShapeDtypeStruct((_SELR, _C), jnp.int32),
            jax.ShapeDtypeStruct((1, 1), jnp.int32),
        ],
        scratch_shapes=[
            pltpu.VMEM((_R, _C), jnp.float32),
            pltpu.VMEM((_R, _C), jnp.float32),
        ],
    )(thr, planes[0], planes[1], planes[2], planes[3], s)

    sel = sel_m.reshape(-1)[:_MOUT]
    num = jnp.minimum(num_m[0, 0], jnp.asarray(max_output_size, jnp.int32))
    return (sel, num)
